# baseline (speedup 1.0000x reference)
import jax
import jax.numpy as jnp
from jax import lax
from jax.experimental import pallas as pl
from jax.experimental.pallas import tpu as pltpu
from jax.experimental.pallas import tpu_sc as plsc

NUM_SEG = 256
N_NODES = 100000
N_EDGES = 6400000

NC = 2
NS = 16
L = 16
NW = NC * NS

EDGES_PER_TILE = N_EDGES // NW
ECHUNK = 10000
N_EPAIR = EDGES_PER_TILE // (2 * ECHUNK)
EUNROLL = 5
EVEC_ITERS = ECHUNK // (L * EUNROLL)
VECS_PER_TILE = EDGES_PER_TILE // L
VECS_PER_CHUNK = ECHUNK // L

DEC_PAD = 32
DEC_BUF = VECS_PER_TILE + 28
DVEC_ITERS = (VECS_PER_TILE + L) // L
MAX_STRADDLE = 272

NODES_PAD = 102400
NODES_PER_TILE = NODES_PAD // NW
NUNROLL = 4
NVEC_ITERS = NODES_PER_TILE // (L * NUNROLL)

TAB_ROWS = 264
TAB_WORDS = TAB_ROWS * L


def _reduce_table(tab, partials, base):
    lane = lax.iota(jnp.int32, L)

    @plsc.parallel_loop(0, NUM_SEG // L)
    def _group(g):
        vec = jnp.zeros((L,), jnp.float32)
        for s2 in range(L):
            row = tab[pl.ds((g * L + s2) * L, L)]
            vec = jnp.where(lane == s2, jnp.sum(row), vec)
        partials[pl.ds(base + g * L, L)] = vec


def _sc_body(qg_hbm, qt_hbm, es_hbm, dec_hbm, xg_hbm, xt_hbm, ns_hbm, out_hbm,
             qg_buf, qt_buf, dec_buf, segv_buf, slist, se_buf, sg_buf, st_buf,
             xg_buf, xt_buf, ns_buf,
             etab, ntab, ctab, partials,
             esem0, esem1, nsem, ssem):
    wid = lax.axis_index("s") * NC + lax.axis_index("c")
    lane = lax.iota(jnp.int32, L)
    ones = jnp.ones((L,), jnp.float32)

    ebase = wid * EDGES_PER_TILE
    esems = (esem0, esem1)

    def estart(ch, b):
        sl = pl.ds(ebase + ch * ECHUNK, ECHUNK)
        dsl = pl.ds(b * ECHUNK, ECHUNK)
        pltpu.async_copy(qg_hbm.at[sl], qg_buf.at[dsl], esems[b])
        pltpu.async_copy(qt_hbm.at[sl], qt_buf.at[dsl], esems[b])

    def ewait(ch, b):
        sl = pl.ds(ebase + ch * ECHUNK, ECHUNK)
        dsl = pl.ds(b * ECHUNK, ECHUNK)
        pltpu.make_async_copy(qg_hbm.at[sl], qg_buf.at[dsl], esems[b]).wait()
        pltpu.make_async_copy(qt_hbm.at[sl], qt_buf.at[dsl], esems[b]).wait()

    doff = wid * VECS_PER_TILE
    dbase = (doff // 8) * 8
    rem = doff - dbase
    dsl_hbm = pl.ds(dbase, DEC_BUF)
    pltpu.async_copy(dec_hbm.at[dsl_hbm], dec_buf, nsem)

    nbase = wid * NODES_PER_TILE
    nsl = pl.ds(nbase, NODES_PER_TILE)
    pltpu.async_copy(xg_hbm.at[:, nsl], xg_buf, nsem)
    pltpu.async_copy(xt_hbm.at[:, nsl], xt_buf, nsem)
    pltpu.async_copy(ns_hbm.at[nsl], ns_buf, nsem)
    estart(0, 0)
    estart(1, 1)

    @plsc.parallel_loop(0, TAB_ROWS)
    def _z_edge(r):
        etab[pl.ds(r * L, L)] = jnp.zeros((L,), jnp.float32)

    @plsc.parallel_loop(0, TAB_ROWS)
    def _z_node(r):
        z = jnp.zeros((L,), jnp.float32)
        ntab[pl.ds(r * L, L)] = z
        ctab[pl.ds(r * L, L)] = z

    pltpu.make_async_copy(dec_hbm.at[dsl_hbm], dec_buf, nsem).wait()
    pltpu.make_async_copy(xg_hbm.at[:, nsl], xg_buf, nsem).wait()
    pltpu.make_async_copy(xt_hbm.at[:, nsl], xt_buf, nsem).wait()
    pltpu.make_async_copy(ns_hbm.at[nsl], ns_buf, nsem).wait()

    def dvec_body(k, cnt):
        va = dec_buf[pl.ds(k * L + rem, L)]
        vb = dec_buf[pl.ds(k * L + rem + 1, L)]
        m = va != vb
        segv_buf[pl.ds(k * L, L)] = jnp.where(m, 257, va)
        vid = lane + k * L
        m = jnp.logical_and(m, vid < VECS_PER_TILE)
        plsc.store_compressed(slist.at[pl.ds(cnt, L)], vid, mask=m)
        add = plsc.all_reduce_population_count(m)
        return cnt + add[0]

    scnt = lax.fori_loop(0, DVEC_ITERS, dvec_body, jnp.int32(0))

    @plsc.parallel_loop(0, NVEC_ITERS)
    def _node(k):
        for u in range(NUNROLL):
            sl = pl.ds((k * NUNROLL + u) * L, L)
            dx = xg_buf[0, sl] - xt_buf[0, sl]
            dy = xg_buf[1, sl] - xt_buf[1, sl]
            dz = xg_buf[2, sl] - xt_buf[2, sl]
            err = dx * dx + dy * dy + dz * dz
            idx = ns_buf[sl] * L + lane
            plsc.addupdate_scatter(ntab, [idx], err)
            plsc.addupdate_scatter(ctab, [idx], ones)

    def pair_body(k, carry):
        ch = k * 2

        def compute(b, ch_):
            vbase = ch_ * VECS_PER_CHUNK

            @plsc.parallel_loop(0, EVEC_ITERS)
            def _vec(j):
                sv = segv_buf[pl.ds(vbase + j * EUNROLL, L)]
                for u in range(EUNROLL):
                    v = j * EUNROLL + u
                    sl = pl.ds(b * ECHUNK + v * L, L)
                    d = qg_buf[sl] - qt_buf[sl]
                    idx = lane + sv[u] * L
                    plsc.addupdate_scatter(etab, [idx], d * d)

        ewait(ch, 0)
        compute(0, ch)

        @pl.when(k < N_EPAIR - 1)
        def _():
            estart(ch + 2, 0)

        ewait(ch + 1, 1)
        compute(1, ch + 1)

        @pl.when(k < N_EPAIR - 1)
        def _():
            estart(ch + 3, 1)

        return carry

    lax.fori_loop(0, N_EPAIR, pair_body, 0)

    def sfire(i, carry):
        sv = slist[pl.ds(i, L)]
        sl = pl.ds((wid * VECS_PER_TILE + sv[0]) * L, L)
        dsl = pl.ds(i * L, L)
        pltpu.async_copy(es_hbm.at[sl], se_buf.at[dsl], ssem)
        pltpu.async_copy(qg_hbm.at[sl], sg_buf.at[dsl], ssem)
        pltpu.async_copy(qt_hbm.at[sl], st_buf.at[dsl], ssem)
        return carry

    lax.fori_loop(0, scnt, sfire, 0)

    def sdrain(i, carry):
        sv = slist[pl.ds(i, L)]
        sl = pl.ds((wid * VECS_PER_TILE + sv[0]) * L, L)
        dsl = pl.ds(i * L, L)
        pltpu.make_async_copy(es_hbm.at[sl], se_buf.at[dsl], ssem).wait()
        pltpu.make_async_copy(qg_hbm.at[sl], sg_buf.at[dsl], ssem).wait()
        pltpu.make_async_copy(qt_hbm.at[sl], st_buf.at[dsl], ssem).wait()
        d = sg_buf[dsl] - st_buf[dsl]
        idx = se_buf[dsl] * L + lane
        plsc.addupdate_scatter(etab, [idx], d * d)
        return carry

    lax.fori_loop(0, scnt, sdrain, 0)

    _reduce_table(etab, partials, 0)
    _reduce_table(ntab, partials, NUM_SEG)
    _reduce_table(ctab, partials, 2 * NUM_SEG)

    pltpu.sync_copy(partials, out_hbm.at[wid])


_SC_SCRATCH = [
    pltpu.VMEM((2 * ECHUNK,), jnp.float32),
    pltpu.VMEM((2 * ECHUNK,), jnp.float32),
    pltpu.VMEM((DEC_BUF,), jnp.int32),
    pltpu.VMEM((VECS_PER_TILE + 2 * L,), jnp.int32),
    pltpu.VMEM((MAX_STRADDLE,), jnp.int32),
    pltpu.VMEM((MAX_STRADDLE * L,), jnp.int32),
    pltpu.VMEM((MAX_STRADDLE * L,), jnp.float32),
    pltpu.VMEM((MAX_STRADDLE * L,), jnp.float32),
    pltpu.VMEM((3, NODES_PER_TILE), jnp.float32),
    pltpu.VMEM((3, NODES_PER_TILE), jnp.float32),
    pltpu.VMEM((NODES_PER_TILE,), jnp.int32),
    pltpu.VMEM((TAB_WORDS,), jnp.float32),
    pltpu.VMEM((TAB_WORDS,), jnp.float32),
    pltpu.VMEM((TAB_WORDS,), jnp.float32),
    pltpu.VMEM((3 * NUM_SEG,), jnp.float32),
    pltpu.SemaphoreType.DMA,
    pltpu.SemaphoreType.DMA,
    pltpu.SemaphoreType.DMA,
    pltpu.SemaphoreType.DMA,
]

_sc_partials = pl.kernel(
    _sc_body,
    out_type=jax.ShapeDtypeStruct((NW, 3 * NUM_SEG), jnp.float32),
    mesh=plsc.VectorSubcoreMesh(core_axis_name="c", subcore_axis_name="s"),
    scratch_types=_SC_SCRATCH,
    compiler_params=pltpu.CompilerParams(needs_layout_passes=False),
)


def _fin_body(p_ref, o_ref):
    p = p_ref[...]
    col = jnp.sum(p, axis=0, keepdims=True)
    e = col[:, 0:NUM_SEG]
    n = col[:, NUM_SEG:2 * NUM_SEG]
    c = col[:, 2 * NUM_SEG:3 * NUM_SEG]
    rmsd_m = jnp.sum(jnp.sqrt(n / jnp.maximum(c, 1.0))) / NUM_SEG
    norm_m = jnp.sum(jnp.sqrt(e)) / NUM_SEG
    lanes = lax.broadcasted_iota(jnp.int32, (1, 128), 1)
    o_ref[...] = jnp.where(lanes == 0, rmsd_m,
                           jnp.where(lanes == 1, norm_m, 0.0))


_finalize = pl.pallas_call(
    _fin_body,
    out_shape=jax.ShapeDtypeStruct((1, 128), jnp.float32),
)


@jax.jit
def kernel(x_gen, x_true, node_seg, q_gen, q_true, edge_seg):
    es = edge_seg.astype(jnp.int32)
    ns = node_seg.astype(jnp.int32)
    dec = jnp.pad(es[::L], (0, DEC_PAD), constant_values=-1)
    pad = NODES_PAD - N_NODES
    xg = jnp.pad(x_gen.T, ((0, 0), (0, pad)))
    xt = jnp.pad(x_true.T, ((0, 0), (0, pad)))
    nsp = jnp.pad(ns, (0, pad), constant_values=NUM_SEG)
    partials = _sc_partials(q_gen, q_true, es, dec, xg, xt, nsp)
    out = _finalize(partials)
    return out[0, :2]


# speedup vs baseline: 2.3794x; 2.3794x over previous
import jax
import jax.numpy as jnp
from jax import lax
from jax.experimental import pallas as pl
from jax.experimental.pallas import tpu as pltpu
from jax.experimental.pallas import tpu_sc as plsc

NUM_SEG = 256
N_NODES = 100000
N_EDGES = 6400000

NC = 2
NS = 16
L = 16
NW = NC * NS

EDGES_PER_TILE = N_EDGES // NW
ECHUNK = 10000
N_EPAIR = EDGES_PER_TILE // (2 * ECHUNK)
EUNROLL = 5
EVEC_ITERS = ECHUNK // (L * EUNROLL)

NODES_PAD = 102400
NODES_PER_TILE = NODES_PAD // NW
NUNROLL = 4
NVEC_ITERS = NODES_PER_TILE // (L * NUNROLL)

TAB_ROWS = 264
TAB_WORDS = TAB_ROWS * L


def _reduce_table(tab, n_phase, partials, base):
    lane = lax.iota(jnp.int32, L)

    @plsc.parallel_loop(0, NUM_SEG // L)
    def _group(g):
        vec = jnp.zeros((L,), jnp.float32)
        for s2 in range(L):
            row = tab[pl.ds((g * L + s2) * L, L)]
            for p in range(1, n_phase):
                row = row + tab[pl.ds(p * TAB_WORDS + (g * L + s2) * L, L)]
            vec = jnp.where(lane == s2, jnp.sum(row), vec)
        partials[pl.ds(base + g * L, L)] = vec


def _sc_body(qg_hbm, qt_hbm, es_hbm, xg_hbm, xt_hbm, ns_hbm, out_hbm,
             qg_buf, qt_buf, es_buf, xg_buf, xt_buf, ns_buf,
             etab, ntab, ctab, partials,
             esem0, esem1, nsem):
    wid = lax.axis_index("s") * NC + lax.axis_index("c")
    lane = lax.iota(jnp.int32, L)
    ones = jnp.ones((L,), jnp.float32)

    ebase = wid * EDGES_PER_TILE
    esems = (esem0, esem1)

    def estart(ch, b):
        sl = pl.ds(ebase + ch * ECHUNK, ECHUNK)
        dsl = pl.ds(b * ECHUNK, ECHUNK)
        pltpu.async_copy(qg_hbm.at[sl], qg_buf.at[dsl], esems[b])
        pltpu.async_copy(qt_hbm.at[sl], qt_buf.at[dsl], esems[b])
        pltpu.async_copy(es_hbm.at[sl], es_buf.at[dsl], esems[b])

    def ewait(ch, b):
        sl = pl.ds(ebase + ch * ECHUNK, ECHUNK)
        dsl = pl.ds(b * ECHUNK, ECHUNK)
        pltpu.make_async_copy(qg_hbm.at[sl], qg_buf.at[dsl], esems[b]).wait()
        pltpu.make_async_copy(qt_hbm.at[sl], qt_buf.at[dsl], esems[b]).wait()
        pltpu.make_async_copy(es_hbm.at[sl], es_buf.at[dsl], esems[b]).wait()

    nbase = wid * NODES_PER_TILE
    nsl = pl.ds(nbase, NODES_PER_TILE)
    pltpu.async_copy(xg_hbm.at[:, nsl], xg_buf, nsem)
    pltpu.async_copy(xt_hbm.at[:, nsl], xt_buf, nsem)
    pltpu.async_copy(ns_hbm.at[nsl], ns_buf, nsem)
    estart(0, 0)
    estart(1, 1)

    @plsc.parallel_loop(0, TAB_ROWS)
    def _z_edge(r):
        etab[pl.ds(r * L, L)] = jnp.zeros((L,), jnp.float32)

    @plsc.parallel_loop(0, TAB_ROWS)
    def _z_node(r):
        z = jnp.zeros((L,), jnp.float32)
        ntab[pl.ds(r * L, L)] = z
        ctab[pl.ds(r * L, L)] = z

    pltpu.make_async_copy(xg_hbm.at[:, nsl], xg_buf, nsem).wait()
    pltpu.make_async_copy(xt_hbm.at[:, nsl], xt_buf, nsem).wait()
    pltpu.make_async_copy(ns_hbm.at[nsl], ns_buf, nsem).wait()

    @plsc.parallel_loop(0, NVEC_ITERS)
    def _node(k):
        for u in range(NUNROLL):
            sl = pl.ds((k * NUNROLL + u) * L, L)
            dx = xg_buf[0, sl] - xt_buf[0, sl]
            dy = xg_buf[1, sl] - xt_buf[1, sl]
            dz = xg_buf[2, sl] - xt_buf[2, sl]
            err = dx * dx + dy * dy + dz * dz
            idx = ns_buf[sl] * L + lane
            plsc.addupdate_scatter(ntab, [idx], err)
            plsc.addupdate_scatter(ctab, [idx], ones)

    def pair_body(k, carry):
        ch = k * 2

        def compute(b):
            @plsc.parallel_loop(0, EVEC_ITERS)
            def _vec(j):
                for u in range(EUNROLL):
                    sl = pl.ds(b * ECHUNK + (j * EUNROLL + u) * L, L)
                    d = qg_buf[sl] - qt_buf[sl]
                    idx = es_buf[sl] * L + lane
                    plsc.addupdate_scatter(etab, [idx], d * d)

        ewait(ch, 0)
        compute(0)

        @pl.when(k < N_EPAIR - 1)
        def _():
            estart(ch + 2, 0)

        ewait(ch + 1, 1)
        compute(1)

        @pl.when(k < N_EPAIR - 1)
        def _():
            estart(ch + 3, 1)

        return carry

    lax.fori_loop(0, N_EPAIR, pair_body, 0)

    _reduce_table(etab, 1, partials, 0)
    _reduce_table(ntab, 1, partials, NUM_SEG)
    _reduce_table(ctab, 1, partials, 2 * NUM_SEG)

    pltpu.sync_copy(partials, out_hbm.at[wid])


_SC_SCRATCH = [
    pltpu.VMEM((2 * ECHUNK,), jnp.float32),
    pltpu.VMEM((2 * ECHUNK,), jnp.float32),
    pltpu.VMEM((2 * ECHUNK,), jnp.int32),
    pltpu.VMEM((3, NODES_PER_TILE), jnp.float32),
    pltpu.VMEM((3, NODES_PER_TILE), jnp.float32),
    pltpu.VMEM((NODES_PER_TILE,), jnp.int32),
    pltpu.VMEM((TAB_WORDS,), jnp.float32),
    pltpu.VMEM((TAB_WORDS,), jnp.float32),
    pltpu.VMEM((TAB_WORDS,), jnp.float32),
    pltpu.VMEM((3 * NUM_SEG,), jnp.float32),
    pltpu.SemaphoreType.DMA,
    pltpu.SemaphoreType.DMA,
    pltpu.SemaphoreType.DMA,
]

_sc_partials = pl.kernel(
    _sc_body,
    out_type=jax.ShapeDtypeStruct((NW, 3 * NUM_SEG), jnp.float32),
    mesh=plsc.VectorSubcoreMesh(core_axis_name="c", subcore_axis_name="s"),
    scratch_types=_SC_SCRATCH,
    compiler_params=pltpu.CompilerParams(needs_layout_passes=False),
)


def _fin_body(p_ref, o_ref):
    p = p_ref[...]
    col = jnp.sum(p, axis=0, keepdims=True)
    e = col[:, 0:NUM_SEG]
    n = col[:, NUM_SEG:2 * NUM_SEG]
    c = col[:, 2 * NUM_SEG:3 * NUM_SEG]
    rmsd_m = jnp.sum(jnp.sqrt(n / jnp.maximum(c, 1.0))) / NUM_SEG
    norm_m = jnp.sum(jnp.sqrt(e)) / NUM_SEG
    lanes = lax.broadcasted_iota(jnp.int32, (1, 128), 1)
    o_ref[...] = jnp.where(lanes == 0, rmsd_m,
                           jnp.where(lanes == 1, norm_m, 0.0))


_finalize = pl.pallas_call(
    _fin_body,
    out_shape=jax.ShapeDtypeStruct((1, 128), jnp.float32),
)


@jax.jit
def kernel(x_gen, x_true, node_seg, q_gen, q_true, edge_seg):
    es = edge_seg.astype(jnp.int32)
    ns = node_seg.astype(jnp.int32)
    pad = NODES_PAD - N_NODES
    xg = jnp.pad(x_gen.T, ((0, 0), (0, pad)))
    xt = jnp.pad(x_true.T, ((0, 0), (0, pad)))
    nsp = jnp.pad(ns, (0, pad), constant_values=NUM_SEG)
    partials = _sc_partials(q_gen, q_true, es, xg, xt, nsp)
    out = _finalize(partials)
    return out[0, :2]
